# Initial kernel scaffold; baseline (speedup 1.0000x reference)
import jax
import jax.numpy as jnp
from jax.experimental import pallas as pl


def kernel(entity, edge, type, relation_embedding):
    raise NotImplementedError("write your pallas kernel here")


# speedup vs baseline: 1.5053x; 1.5053x over previous
import functools

import jax
import jax.numpy as jnp
from jax import lax
from jax.experimental import pallas as pl
from jax.experimental.pallas import tpu as pltpu
from jax.experimental.pallas import tpu_sc as plsc

N_EDGES = 160000
D = 256
LANES = 16

_info = plsc.get_sparse_core_info()
NC = _info.num_cores
NS = _info.num_subcores
NW = NC * NS

B = 128
EW_BLOCKS = -(-N_EDGES // (NW * B))
EW = EW_BLOCKS * B
N_PAD = EW * NW


def _body(entity_hbm, src_hbm, dst_hbm, typ_hbm, rel_hbm, out_hbm,
          src_v, dst_v, typ_v, s_v, o_v, r_v, ob_v, sem):
    wid = lax.axis_index("s") * NC + lax.axis_index("c")

    def block(g, _):
        base = wid * EW + g * B
        pltpu.sync_copy(src_hbm.at[pl.ds(base, B)], src_v)
        pltpu.sync_copy(dst_hbm.at[pl.ds(base, B)], dst_v)
        pltpu.sync_copy(typ_hbm.at[pl.ds(base, B)], typ_v)
        c1 = pltpu.async_copy(entity_hbm.at[src_v], s_v, sem)
        c2 = pltpu.async_copy(entity_hbm.at[dst_v], o_v, sem)
        c3 = pltpu.async_copy(rel_hbm.at[typ_v], r_v, sem)
        c1.wait()
        c2.wait()
        c3.wait()
        lane = lax.iota(jnp.int32, LANES)

        dnums = lax.GatherDimensionNumbers(
            offset_dims=(), collapsed_slice_dims=(0,), start_index_map=(0,))

        def permute(v, idx):
            return lax.gather(v, idx[:, None], dnums, (1,),
                              mode=lax.GatherScatterMode.PROMISE_IN_BOUNDS)

        def lanesum(v):
            for k in (8, 4, 2, 1):
                v = v + permute(v, lane ^ k)
            return v

        def group(grp, _):
            def edge(k, res):
                b = grp * LANES + k
                acc = jnp.zeros((LANES,), jnp.float32)
                for j in range(D // LANES):
                    ds = pl.ds(j * LANES, LANES)
                    acc = acc + s_v[b, ds] * r_v[b, ds] * o_v[b, ds]
                return jnp.where(lane == k, lanesum(acc), res)

            res = lax.fori_loop(0, LANES, edge,
                                jnp.zeros((LANES,), jnp.float32))
            ob_v[pl.ds(grp * LANES, LANES)] = res
            return _

        lax.fori_loop(0, B // LANES, group, None)
        pltpu.sync_copy(ob_v, out_hbm.at[pl.ds(base, B)])
        return _

    lax.fori_loop(0, EW_BLOCKS, block, None)


@jax.jit
def _distmult(entity, src, dst, typ, rel):
    k = functools.partial(
        pl.kernel,
        mesh=plsc.VectorSubcoreMesh(core_axis_name="c", subcore_axis_name="s"),
        out_type=jax.ShapeDtypeStruct((N_PAD,), jnp.float32),
        compiler_params=pltpu.CompilerParams(use_tc_tiling_on_sc=False),
        scratch_types=[
            pltpu.VMEM((B,), jnp.int32),
            pltpu.VMEM((B,), jnp.int32),
            pltpu.VMEM((B,), jnp.int32),
            pltpu.VMEM((B, D), jnp.float32),
            pltpu.VMEM((B, D), jnp.float32),
            pltpu.VMEM((B, D), jnp.float32),
            pltpu.VMEM((B,), jnp.float32),
            pltpu.SemaphoreType.DMA,
        ],
    )(_body)
    return k(entity, src, dst, typ, rel)


def kernel(entity, edge, type, relation_embedding):
    pad = N_PAD - N_EDGES
    src = jnp.pad(edge[:, 0].astype(jnp.int32), (0, pad))
    dst = jnp.pad(edge[:, 1].astype(jnp.int32), (0, pad))
    typ = jnp.pad(type.astype(jnp.int32), (0, pad))
    out = _distmult(entity, src, dst, typ, relation_embedding)
    return out[:N_EDGES]


# speedup vs baseline: 1.9848x; 1.3185x over previous
import functools

import jax
import jax.numpy as jnp
from jax import lax
from jax.experimental import pallas as pl
from jax.experimental.pallas import tpu as pltpu
from jax.experimental.pallas import tpu_sc as plsc

N_EDGES = 160000
D = 256
LANES = 16

_info = plsc.get_sparse_core_info()
NC = _info.num_cores
NS = _info.num_subcores
NW = NC * NS

B = 64
NBUF = 2
EW_BLOCKS = -(-N_EDGES // (NW * B * NBUF)) * NBUF
EW = EW_BLOCKS * B
N_PAD = EW * NW


def _body(entity_hbm, src_hbm, dst_hbm, typ_hbm, rel_hbm, out_hbm,
          src_v, dst_v, typ_v, s_v, o_v, r_v, ob_v, sem):
    wid = lax.axis_index("s") * NC + lax.axis_index("c")
    first = wid * EW
    last = first + (EW_BLOCKS - 1) * B

    def start(slot, base):
        pltpu.sync_copy(src_hbm.at[pl.ds(base, B)], src_v.at[slot])
        pltpu.sync_copy(dst_hbm.at[pl.ds(base, B)], dst_v.at[slot])
        pltpu.sync_copy(typ_hbm.at[pl.ds(base, B)], typ_v.at[slot])
        pltpu.async_copy(entity_hbm.at[src_v.at[slot]], s_v.at[slot], sem.at[slot])
        pltpu.async_copy(entity_hbm.at[dst_v.at[slot]], o_v.at[slot], sem.at[slot])
        pltpu.async_copy(rel_hbm.at[typ_v.at[slot]], r_v.at[slot], sem.at[slot])

    def drain(slot):
        pltpu.make_async_copy(entity_hbm.at[src_v.at[slot]], s_v.at[slot],
                              sem.at[slot]).wait()
        pltpu.make_async_copy(entity_hbm.at[dst_v.at[slot]], o_v.at[slot],
                              sem.at[slot]).wait()
        pltpu.make_async_copy(rel_hbm.at[typ_v.at[slot]], r_v.at[slot],
                              sem.at[slot]).wait()

    lane = lax.iota(jnp.int32, LANES)
    dnums = lax.GatherDimensionNumbers(
        offset_dims=(), collapsed_slice_dims=(0,), start_index_map=(0,))

    def permute(v, idx):
        return lax.gather(v, idx[:, None], dnums, (1,),
                          mode=lax.GatherScatterMode.PROMISE_IN_BOUNDS)

    def lanesum(v):
        for k in (8, 4, 2, 1):
            v = v + permute(v, lane ^ k)
        return v

    def compute(slot, base):
        sb, ob, rb, outb = s_v.at[slot], o_v.at[slot], r_v.at[slot], ob_v.at[slot]

        def group(grp, _):
            def edge(k, res):
                b = grp * LANES + k
                acc = jnp.zeros((LANES,), jnp.float32)
                for j in range(D // LANES):
                    ds = pl.ds(j * LANES, LANES)
                    acc = acc + sb[b, ds] * rb[b, ds] * ob[b, ds]
                return jnp.where(lane == k, lanesum(acc), res)

            res = lax.fori_loop(0, LANES, edge,
                                jnp.zeros((LANES,), jnp.float32))
            outb[pl.ds(grp * LANES, LANES)] = res
            return _

        lax.fori_loop(0, B // LANES, group, None)
        pltpu.sync_copy(outb, out_hbm.at[pl.ds(base, B)])

    start(0, first)

    def pair(gg, _):
        g0 = gg * NBUF
        for sl in range(NBUF):
            g = g0 + sl
            base = first + g * B
            nxt = jnp.minimum(base + B, last)
            start((sl + 1) % NBUF, nxt)
            drain(sl)
            compute(sl, base)
        return _

    lax.fori_loop(0, EW_BLOCKS // NBUF, pair, None)
    drain(0)


@jax.jit
def _distmult(entity, src, dst, typ, rel):
    k = functools.partial(
        pl.kernel,
        mesh=plsc.VectorSubcoreMesh(core_axis_name="c", subcore_axis_name="s"),
        out_type=jax.ShapeDtypeStruct((N_PAD,), jnp.float32),
        compiler_params=pltpu.CompilerParams(use_tc_tiling_on_sc=False),
        scratch_types=[
            pltpu.VMEM((NBUF, B), jnp.int32),
            pltpu.VMEM((NBUF, B), jnp.int32),
            pltpu.VMEM((NBUF, B), jnp.int32),
            pltpu.VMEM((NBUF, B, D), jnp.float32),
            pltpu.VMEM((NBUF, B, D), jnp.float32),
            pltpu.VMEM((NBUF, B, D), jnp.float32),
            pltpu.VMEM((NBUF, B), jnp.float32),
            pltpu.SemaphoreType.DMA((NBUF,)),
        ],
    )(_body)
    return k(entity, src, dst, typ, rel)


def kernel(entity, edge, type, relation_embedding):
    pad = N_PAD - N_EDGES
    src = jnp.pad(edge[:, 0].astype(jnp.int32), (0, pad))
    dst = jnp.pad(edge[:, 1].astype(jnp.int32), (0, pad))
    typ = jnp.pad(type.astype(jnp.int32), (0, pad))
    out = _distmult(entity, src, dst, typ, relation_embedding)
    return out[:N_EDGES]
